# Initial kernel scaffold; baseline (speedup 1.0000x reference)
import jax
import jax.numpy as jnp
from jax.experimental import pallas as pl


def kernel(coordinates, edge_src, edge_dst):
    raise NotImplementedError("write your pallas kernel here")


# speedup vs baseline: 10.6042x; 10.6042x over previous
import functools

import jax
import jax.numpy as jnp
from jax import lax
from jax.experimental import pallas as pl
from jax.experimental.pallas import tpu as pltpu
from jax.experimental.pallas import tpu_sc as plsc

CUTOFF = 0.5
NC = 2
NS = 16
NW = NC * NS

C = 2048
IW = 128
KS = C // IW
GROUPS = C // 16


def _edge_body(n_chunks,
               x_hbm, y_hbm, z_hbm, src_hbm, dst_hbm,
               vec_hbm, dist_hbm, sw_hbm, mask_hbm,
               x_sp, y_sp, z_sp, idx_s, idx_d,
               sxb, syb, szb, dxb, dyb, dzb,
               vout, dout, sout, mout, sem):
    cid = lax.axis_index("c")
    sid = lax.axis_index("s")
    wid = sid * NC + cid

    @pl.when(sid == 0)
    def _():
        pltpu.sync_copy(x_hbm, x_sp)
        pltpu.sync_copy(y_hbm, y_sp)
        pltpu.sync_copy(z_hbm, z_sp)

    plsc.subcore_barrier()

    lane = lax.iota(jnp.int32, 16)
    n_mine = (n_chunks // NW) + jnp.where(wid < (n_chunks % NW), 1, 0)

    def chunk_body(t, carry):
        ci = t * NW + wid
        ebase = ci * C

        pltpu.sync_copy(src_hbm.at[pl.ds(ebase, C)], idx_s)
        pltpu.sync_copy(dst_hbm.at[pl.ds(ebase, C)], idx_d)

        descs = []
        for k in range(KS):
            sl = pl.ds(k * IW, IW)
            isl = idx_s.at[sl]
            idl = idx_d.at[sl]
            descs.append(pltpu.async_copy(x_sp.at[isl], sxb.at[sl], sem))
            descs.append(pltpu.async_copy(y_sp.at[isl], syb.at[sl], sem))
            descs.append(pltpu.async_copy(z_sp.at[isl], szb.at[sl], sem))
            descs.append(pltpu.async_copy(x_sp.at[idl], dxb.at[sl], sem))
            descs.append(pltpu.async_copy(y_sp.at[idl], dyb.at[sl], sem))
            descs.append(pltpu.async_copy(z_sp.at[idl], dzb.at[sl], sem))
        for d in descs:
            d.wait()

        def grp(j, carry2):
            e16 = pl.ds(j * 16, 16)
            vx = dxb[e16] - sxb[e16]
            vy = dyb[e16] - syb[e16]
            vz = dzb[e16] - szb[e16]
            n2 = vx * vx + vy * vy + vz * vz
            n2c = jnp.maximum(n2, 1e-30)
            ib = 0x5F3759DF - (plsc.bitcast(n2c, jnp.int32) >> 1)
            y = plsc.bitcast(ib, jnp.float32)
            y = y * (1.5 - 0.5 * n2c * y * y)
            y = y * (1.5 - 0.5 * n2c * y * y)
            y = y * (1.5 - 0.5 * n2c * y * y)
            dist = n2 * y
            m = dist < CUTOFF
            xc = jnp.minimum(dist * (1.0 / CUTOFF), 1.0)
            t_ = (xc - 0.5) * 3.14159265358979
            t2 = t_ * t_
            p = 1.0 / 362880.0
            p = p * t2 - 1.0 / 5040.0
            p = p * t2 + 1.0 / 120.0
            p = p * t2 - 1.0 / 6.0
            p = p * t2 + 1.0
            s = 0.5 - 0.5 * (t_ * p)
            sw = jnp.where(m, s, 0.0)
            mf = jnp.where(m, 1.0, 0.0)

            i3 = (j * 16 + lane) * 3
            plsc.store_scatter(vout, [i3], vx)
            plsc.store_scatter(vout, [i3 + 1], vy)
            plsc.store_scatter(vout, [i3 + 2], vz)
            dout[e16] = dist
            sout[e16] = sw
            mout[e16] = mf
            return carry2

        lax.fori_loop(0, GROUPS, grp, 0)

        pltpu.sync_copy(vout, vec_hbm.at[pl.ds(ebase * 3, 3 * C)])
        pltpu.sync_copy(dout, dist_hbm.at[pl.ds(ebase, C)])
        pltpu.sync_copy(sout, sw_hbm.at[pl.ds(ebase, C)])
        pltpu.sync_copy(mout, mask_hbm.at[pl.ds(ebase, C)])
        return carry

    lax.fori_loop(0, n_mine, chunk_body, 0)


@jax.jit
def _run(x, y, z, src, dst):
    n_nodes = x.shape[0]
    n_edges = src.shape[0]
    assert n_edges % C == 0
    n_chunks = n_edges // C

    mesh = plsc.VectorSubcoreMesh(core_axis_name="c", subcore_axis_name="s")
    f32 = jnp.float32
    kern = pl.kernel(
        functools.partial(_edge_body, n_chunks),
        out_type=[
            jax.ShapeDtypeStruct((3 * n_edges,), f32),
            jax.ShapeDtypeStruct((n_edges,), f32),
            jax.ShapeDtypeStruct((n_edges,), f32),
            jax.ShapeDtypeStruct((n_edges,), f32),
        ],
        mesh=mesh,
        compiler_params=pltpu.CompilerParams(needs_layout_passes=False),
        scratch_types=[
            pltpu.VMEM_SHARED((n_nodes,), f32),
            pltpu.VMEM_SHARED((n_nodes,), f32),
            pltpu.VMEM_SHARED((n_nodes,), f32),
            pltpu.VMEM((C,), jnp.int32),
            pltpu.VMEM((C,), jnp.int32),
            pltpu.VMEM((C,), f32),
            pltpu.VMEM((C,), f32),
            pltpu.VMEM((C,), f32),
            pltpu.VMEM((C,), f32),
            pltpu.VMEM((C,), f32),
            pltpu.VMEM((C,), f32),
            pltpu.VMEM((3 * C,), f32),
            pltpu.VMEM((C,), f32),
            pltpu.VMEM((C,), f32),
            pltpu.VMEM((C,), f32),
            pltpu.SemaphoreType.DMA,
        ],
    )
    return kern(x, y, z, src, dst)


def kernel(coordinates, edge_src, edge_dst):
    n_edges = edge_src.shape[0]
    x = coordinates[:, 0]
    y = coordinates[:, 1]
    z = coordinates[:, 2]
    vec_flat, dist, sw, maskf = _run(x, y, z, edge_src, edge_dst)
    return (vec_flat.reshape(n_edges, 3), dist, sw,
            maskf.astype(jnp.bool_))
